# baseline (device time: 223551 ns/iter reference)
import jax
import jax.numpy as jnp
from jax import lax
from jax.experimental import pallas as pl
from jax.experimental.pallas import tpu as pltpu

M, N = 16384, 2048
MH, NH = M // 2, N // 2

SIZES = [128, 256] + [608] * 12 + [256, 128, 64, 64]
assert sum(SIZES) == MH
C = len(SIZES)
OFFS = [sum(SIZES[:i]) for i in range(C)]
MAXCH = max(SIZES)


def kernel(x):
    def body(x_ref, out_ref, send_ref, recvx_ref,
             stage_p, stage_m, lp_sems, lm_sems, store_sems,
             sx_send, sx_recv, sy_send, sy_recv):
        mx = lax.axis_index("x")
        my = lax.axis_index("y")
        rows0 = my * MH
        mcol = mx * NH
        pcol = (1 - mx) * NH

        barrier_sem = pltpu.get_barrier_semaphore()
        for nbr in ((1 - mx, my), (mx, 1 - my)):
            pl.semaphore_signal(
                barrier_sem, inc=1,
                device_id=nbr, device_id_type=pl.DeviceIdType.MESH,
            )
        pl.semaphore_wait(barrier_sem, 2)

        def load(c, stage, sems, col0):
            return pltpu.make_async_copy(
                x_ref.at[0, pl.ds(rows0 + OFFS[c], SIZES[c]), pl.ds(col0, NH)],
                stage.at[c % 2, pl.ds(0, SIZES[c]), :], sems.at[c % 2])

        loads_p = [load(c, stage_p, lp_sems, pcol) for c in range(C)]
        loads_m = [load(c, stage_m, lm_sems, mcol) for c in range(C)]
        rdmas_x = []
        loads_p[0].start()
        for c in range(C):
            off, sz = OFFS[c], SIZES[c]
            if c + 1 < C:
                loads_p[c + 1].start()
            loads_p[c].wait()
            send_ref[pl.ds(off, sz), :] = (
                stage_p[c % 2, pl.ds(0, sz), :].astype(jnp.bfloat16))
            r = pltpu.make_async_remote_copy(
                src_ref=send_ref.at[pl.ds(off, sz), :],
                dst_ref=recvx_ref.at[pl.ds(off, sz), :],
                send_sem=sx_send.at[c], recv_sem=sx_recv.at[c],
                device_id=(1 - mx, my), device_id_type=pl.DeviceIdType.MESH,
            )
            r.start()
            rdmas_x.append(r)

        stores = []
        rdmas_y = []
        loads_m[0].start()
        for c in range(C):
            off, sz = OFFS[c], SIZES[c]
            if c + 1 < C:
                loads_m[c + 1].start()
            loads_m[c].wait()
            rdmas_x[c].wait_recv()
            recvx_ref[pl.ds(off, sz), :] = (
                recvx_ref[pl.ds(off, sz), :]
                + stage_m[c % 2, pl.ds(0, sz), :].astype(jnp.bfloat16))
            st = pltpu.make_async_copy(
                recvx_ref.at[pl.ds(off, sz), :],
                out_ref.at[pl.ds(rows0 + off, sz), :],
                store_sems.at[c],
            )
            st.start()
            stores.append(st)
            ry = pltpu.make_async_remote_copy(
                src_ref=recvx_ref.at[pl.ds(off, sz), :],
                dst_ref=out_ref.at[pl.ds(rows0 + off, sz), :],
                send_sem=sy_send.at[c], recv_sem=sy_recv.at[c],
                device_id=(mx, 1 - my), device_id_type=pl.DeviceIdType.MESH,
            )
            ry.start()
            rdmas_y.append(ry)

        for c in range(C):
            rdmas_x[c].wait_send()
            stores[c].wait()
            rdmas_y[c].wait()

    return pl.pallas_call(
        body,
        out_shape=jax.ShapeDtypeStruct((M, NH), jnp.bfloat16),
        in_specs=[pl.BlockSpec(memory_space=pl.ANY)],
        out_specs=pl.BlockSpec(memory_space=pl.ANY),
        scratch_shapes=[
            pltpu.VMEM((MH, NH), jnp.bfloat16),
            pltpu.VMEM((MH, NH), jnp.bfloat16),
            pltpu.VMEM((2, MAXCH, NH), jnp.float32),
            pltpu.VMEM((2, MAXCH, NH), jnp.float32),
            pltpu.SemaphoreType.DMA((2,)),
            pltpu.SemaphoreType.DMA((2,)),
            pltpu.SemaphoreType.DMA((C,)),
            pltpu.SemaphoreType.DMA((C,)),
            pltpu.SemaphoreType.DMA((C,)),
            pltpu.SemaphoreType.DMA((C,)),
            pltpu.SemaphoreType.DMA((C,)),
        ],
        compiler_params=pltpu.CompilerParams(
            collective_id=0, vmem_limit_bytes=63 * 1024 * 1024),
    )(x)
